# baseline (speedup 1.0000x reference)
import functools

import jax
import jax.numpy as jnp
from jax import lax
from jax.experimental import pallas as pl
from jax.experimental.pallas import tpu as pltpu
from jax.experimental.pallas import tpu_sc as plsc

_NC = 2
_NS = 16
_NW = _NC * _NS
_C = 128


def _mesh():
    return plsc.VectorSubcoreMesh(core_axis_name="c", subcore_axis_name="s")


def _sc_segment_sum(m, src3, dst3, n_pad):
    n, h = m.shape
    nchunk = src3.shape[1]
    rows_per_tile = n_pad // _NS
    assert rows_per_tile % _C == 0 and rows_per_tile % 8 == 0

    @functools.partial(
        pl.kernel,
        out_type=jax.ShapeDtypeStruct((_NC, n_pad, h), jnp.float32),
        mesh=_mesh(),
        scratch_types=[
            pltpu.VMEM((nchunk, _C), jnp.int32),
            pltpu.VMEM((nchunk, _C), jnp.int32),
            pltpu.VMEM((_C, h), jnp.float32),
            pltpu.VMEM_SHARED((n_pad, h), jnp.float32),
            pltpu.SemaphoreType.DMA,
        ],
    )
    def seg(m_hbm, src_hbm, dst_hbm, out_hbm, sidx, didx, rows, acc, sem):
        cid = lax.axis_index("c")
        sid = lax.axis_index("s")
        wid = sid * _NC + cid
        row0 = sid * rows_per_tile

        z16 = jnp.zeros((16,), jnp.float32)

        @pl.loop(0, _C)
        def _zero(i):
            for k in range(h // 16):
                rows[i, pl.ds(k * 16, 16)] = z16

        for k in range(rows_per_tile // _C):
            pltpu.sync_copy(rows, acc.at[pl.ds(row0 + k * _C, _C)])
        plsc.subcore_barrier()

        pltpu.sync_copy(src_hbm.at[wid], sidx)
        pltpu.sync_copy(dst_hbm.at[wid], didx)

        @pl.loop(0, nchunk)
        def _edge_chunk(j):
            pltpu.async_copy(m_hbm.at[sidx.at[j]], rows, sem).wait()
            pltpu.sync_copy(rows, acc.at[didx.at[j]], add=True)

        plsc.subcore_barrier()
        pltpu.sync_copy(acc.at[pl.ds(row0, rows_per_tile)],
                        out_hbm.at[cid, pl.ds(row0, rows_per_tile)])

    return seg(m, src3, dst3)


def _sc_gather_pairs(uv, es, ed):
    n, w = uv.shape
    q = es.shape[0]
    rpw = q // _NW

    @functools.partial(
        pl.kernel,
        out_type=(jax.ShapeDtypeStruct((q, w), jnp.float32),
                  jax.ShapeDtypeStruct((q, w), jnp.float32)),
        mesh=_mesh(),
        scratch_types=[
            pltpu.VMEM((rpw,), jnp.int32),
            pltpu.VMEM((rpw,), jnp.int32),
            pltpu.VMEM((rpw, w), jnp.float32),
            pltpu.VMEM((rpw, w), jnp.float32),
            pltpu.SemaphoreType.DMA,
            pltpu.SemaphoreType.DMA,
        ],
    )
    def gat(uv_hbm, es_hbm, ed_hbm, os_hbm, od_hbm, i1, i2, r1, r2, s1, s2):
        cid = lax.axis_index("c")
        sid = lax.axis_index("s")
        base = (sid * _NC + cid) * rpw
        pltpu.sync_copy(es_hbm.at[pl.ds(base, rpw)], i1)
        pltpu.sync_copy(ed_hbm.at[pl.ds(base, rpw)], i2)
        c1 = pltpu.async_copy(uv_hbm.at[i1], r1, s1)
        c2 = pltpu.async_copy(uv_hbm.at[i2], r2, s2)
        c1.wait()
        c2.wait()
        pltpu.sync_copy(r1, os_hbm.at[pl.ds(base, rpw)])
        pltpu.sync_copy(r2, od_hbm.at[pl.ds(base, rpw)])

    return gat(uv, es, ed)


def _tc_node_linear(x, W, b):
    n = x.shape[0]

    def body(x_ref, w_ref, b_ref, o_ref):
        o_ref[...] = jnp.maximum(
            jnp.dot(x_ref[...], w_ref[...],
                    preferred_element_type=jnp.float32) + b_ref[...], 0.0)

    return pl.pallas_call(
        body, out_shape=jax.ShapeDtypeStruct((n, W.shape[1]), jnp.float32),
    )(x, W, b.reshape(1, -1))


def _sage_update(parts_ref, h_ref, wa_ref, ba_ref, hdim):
    n = h_ref.shape[0]
    aggr = parts_ref[0, 0:n] + parts_ref[1, 0:n]
    hp = h_ref[...]
    t = (jnp.dot(aggr, wa_ref[0:hdim], preferred_element_type=jnp.float32)
         + jnp.dot(hp, wa_ref[hdim:2 * hdim],
                   preferred_element_type=jnp.float32)
         + ba_ref[...])
    t = jnp.maximum(t, 0.0)
    nrm = jnp.sqrt(jnp.sum(t * t, axis=1, keepdims=True))
    return t / jnp.maximum(nrm, 1e-12)


def _tc_update_and_msg(parts, h_prev, W_agg, b_agg, W_lin, b_lin):
    n, hdim = h_prev.shape

    def body(p_ref, h_ref, wa_ref, ba_ref, wl_ref, bl_ref, oh_ref, om_ref):
        hnew = _sage_update(p_ref, h_ref, wa_ref, ba_ref, hdim)
        oh_ref[...] = hnew
        om_ref[...] = jnp.maximum(
            jnp.dot(hnew, wl_ref[...],
                    preferred_element_type=jnp.float32) + bl_ref[...], 0.0)

    return pl.pallas_call(
        body,
        out_shape=(jax.ShapeDtypeStruct((n, hdim), jnp.float32),
                   jax.ShapeDtypeStruct((n, hdim), jnp.float32)),
    )(parts, h_prev, W_agg, b_agg.reshape(1, -1), W_lin, b_lin.reshape(1, -1))


def _tc_final_uv(parts, h_prev, W_agg, b_agg, mp_W1, mp_W2):
    n, hdim = h_prev.shape
    o = mp_W2.shape[1]

    def body(p_ref, h_ref, wa_ref, ba_ref, w1_ref, w2_ref, o_ref):
        h2 = _sage_update(p_ref, h_ref, wa_ref, ba_ref, hdim)
        w2 = w2_ref[...]
        wu = jnp.dot(w1_ref[0:hdim], w2, preferred_element_type=jnp.float32)
        wv = jnp.dot(w1_ref[hdim:2 * hdim], w2,
                     preferred_element_type=jnp.float32)
        wc = jnp.concatenate(
            [wu, jnp.zeros((hdim, 16 - o), jnp.float32),
             wv, jnp.zeros((hdim, 128 - 16 - o), jnp.float32)], axis=1)
        o_ref[...] = jnp.dot(h2, wc, preferred_element_type=jnp.float32)

    return pl.pallas_call(
        body, out_shape=jax.ShapeDtypeStruct((n, 128), jnp.float32),
    )(parts, h_prev, W_agg, b_agg.reshape(1, -1), mp_W1, mp_W2)


def _tc_logits(gs, gd, mp_b1, mp_W2, mp_b2):
    q = gs.shape[0]
    o = mp_W2.shape[1]

    def body(gs_ref, gd_ref, b1_ref, w2_ref, b2_ref, o_ref):
        c = jnp.dot(b1_ref[...], w2_ref[...],
                    preferred_element_type=jnp.float32) + b2_ref[...]
        z = gs_ref[:, 0:o] + gd_ref[:, 16:16 + o] + c
        mx = jnp.max(z, axis=1, keepdims=True)
        lse = mx + jnp.log(jnp.sum(jnp.exp(z - mx), axis=1, keepdims=True))
        o_ref[...] = z - lse

    return pl.pallas_call(
        body, out_shape=jax.ShapeDtypeStruct((q, o), jnp.float32),
    )(gs, gd, mp_b1.reshape(1, -1), mp_W2, mp_b2.reshape(1, -1))


def kernel(x, edge_index, batch, eval_edges,
           lin_W0, lin_b0, agg_W0, agg_b0,
           lin_W1, lin_b1, agg_W1, agg_b1,
           mp_W1, mp_b1, mp_W2, mp_b2):
    n = x.shape[0]
    e = edge_index.shape[1]
    src = edge_index[0].astype(jnp.int32)
    dst = edge_index[1].astype(jnp.int32)

    grp = _NW * _C * 16
    e_pad = ((e + grp - 1) // grp) * grp
    pad = e_pad - e
    grp_n = _NS * 128
    n_pad = ((n + grp_n - 1) // grp_n) * grp_n
    if pad:
        src = jnp.concatenate([src, jnp.zeros((pad,), jnp.int32)])
        fill = n + jnp.arange(pad, dtype=jnp.int32) % (n_pad - n)
        dst = jnp.concatenate([dst, fill])
    src3 = src.reshape(_NW, -1, _C)
    dst3 = dst.reshape(_NW, -1, _C)

    m0 = _tc_node_linear(x, lin_W0, lin_b0)
    parts0 = _sc_segment_sum(m0, src3, dst3, n_pad)
    h1, m1 = _tc_update_and_msg(parts0, x, agg_W0, agg_b0, lin_W1, lin_b1)
    parts1 = _sc_segment_sum(m1, src3, dst3, n_pad)
    uv = _tc_final_uv(parts1, h1, agg_W1, agg_b1, mp_W1, mp_W2)
    es = eval_edges[0].astype(jnp.int32)
    ed = eval_edges[1].astype(jnp.int32)
    gs, gd = _sc_gather_pairs(uv, es, ed)
    return _tc_logits(gs, gd, mp_b1, mp_W2, mp_b2)


# speedup vs baseline: 1.4835x; 1.4835x over previous
import functools

import jax
import jax.numpy as jnp
from jax import lax
from jax.experimental import pallas as pl
from jax.experimental.pallas import tpu as pltpu
from jax.experimental.pallas import tpu_sc as plsc

_NC = 2
_NS = 16
_NW = _NC * _NS
_C = 128


def _mesh():
    return plsc.VectorSubcoreMesh(core_axis_name="c", subcore_axis_name="s")


def _sc_segment_sum(m, src3, dst3, n_pad):
    n, h = m.shape
    nchunk = src3.shape[1]
    rows_per_tile = n_pad // _NS
    assert rows_per_tile % _C == 0 and rows_per_tile % 8 == 0

    @functools.partial(
        pl.kernel,
        out_type=jax.ShapeDtypeStruct((_NC, n_pad, h), jnp.float32),
        mesh=_mesh(),
        scratch_types=[
            pltpu.VMEM((nchunk, _C), jnp.int32),
            pltpu.VMEM((nchunk, _C), jnp.int32),
            pltpu.VMEM((_C, h), jnp.float32),
            pltpu.VMEM_SHARED((n_pad, h), jnp.float32),
            pltpu.SemaphoreType.DMA,
        ],
    )
    def seg(m_hbm, src_hbm, dst_hbm, out_hbm, sidx, didx, rows, acc, sem):
        cid = lax.axis_index("c")
        sid = lax.axis_index("s")
        wid = sid * _NC + cid
        row0 = sid * rows_per_tile

        z16 = jnp.zeros((16,), jnp.float32)

        @pl.loop(0, _C)
        def _zero(i):
            for k in range(h // 16):
                rows[i, pl.ds(k * 16, 16)] = z16

        for k in range(rows_per_tile // _C):
            pltpu.sync_copy(rows, acc.at[pl.ds(row0 + k * _C, _C)])
        plsc.subcore_barrier()

        pltpu.sync_copy(src_hbm.at[wid], sidx)
        pltpu.sync_copy(dst_hbm.at[wid], didx)

        @pl.loop(0, nchunk, unroll=1)
        def _edge_chunk(j):
            pltpu.async_copy(m_hbm.at[sidx.at[j]], rows, sem).wait()
            pltpu.sync_copy(rows, acc.at[didx.at[j]], add=True)

        plsc.subcore_barrier()
        pltpu.sync_copy(acc.at[pl.ds(row0, rows_per_tile)],
                        out_hbm.at[cid, pl.ds(row0, rows_per_tile)])

    return seg(m, src3, dst3)


def _sc_gather_pairs(uv, es, ed):
    n, w = uv.shape
    q = es.shape[0]
    rpw = q // _NW

    @functools.partial(
        pl.kernel,
        out_type=(jax.ShapeDtypeStruct((q, w), jnp.float32),
                  jax.ShapeDtypeStruct((q, w), jnp.float32)),
        mesh=_mesh(),
        scratch_types=[
            pltpu.VMEM((rpw,), jnp.int32),
            pltpu.VMEM((rpw,), jnp.int32),
            pltpu.VMEM((rpw, w), jnp.float32),
            pltpu.VMEM((rpw, w), jnp.float32),
            pltpu.SemaphoreType.DMA,
            pltpu.SemaphoreType.DMA,
        ],
    )
    def gat(uv_hbm, es_hbm, ed_hbm, os_hbm, od_hbm, i1, i2, r1, r2, s1, s2):
        cid = lax.axis_index("c")
        sid = lax.axis_index("s")
        base = (sid * _NC + cid) * rpw
        pltpu.sync_copy(es_hbm.at[pl.ds(base, rpw)], i1)
        pltpu.sync_copy(ed_hbm.at[pl.ds(base, rpw)], i2)
        c1 = pltpu.async_copy(uv_hbm.at[i1], r1, s1)
        c2 = pltpu.async_copy(uv_hbm.at[i2], r2, s2)
        c1.wait()
        c2.wait()
        pltpu.sync_copy(r1, os_hbm.at[pl.ds(base, rpw)])
        pltpu.sync_copy(r2, od_hbm.at[pl.ds(base, rpw)])

    return gat(uv, es, ed)


def _tc_node_linear(x, W, b):
    n = x.shape[0]

    def body(x_ref, w_ref, b_ref, o_ref):
        o_ref[...] = jnp.maximum(
            jnp.dot(x_ref[...], w_ref[...],
                    preferred_element_type=jnp.float32) + b_ref[...], 0.0)

    return pl.pallas_call(
        body, out_shape=jax.ShapeDtypeStruct((n, W.shape[1]), jnp.float32),
    )(x, W, b.reshape(1, -1))


def _sage_update(parts_ref, h_ref, wa_ref, ba_ref, hdim):
    n = h_ref.shape[0]
    aggr = parts_ref[0, 0:n] + parts_ref[1, 0:n]
    hp = h_ref[...]
    t = (jnp.dot(aggr, wa_ref[0:hdim], preferred_element_type=jnp.float32)
         + jnp.dot(hp, wa_ref[hdim:2 * hdim],
                   preferred_element_type=jnp.float32)
         + ba_ref[...])
    t = jnp.maximum(t, 0.0)
    nrm = jnp.sqrt(jnp.sum(t * t, axis=1, keepdims=True))
    return t / jnp.maximum(nrm, 1e-12)


def _tc_update_and_msg(parts, h_prev, W_agg, b_agg, W_lin, b_lin):
    n, hdim = h_prev.shape

    def body(p_ref, h_ref, wa_ref, ba_ref, wl_ref, bl_ref, oh_ref, om_ref):
        hnew = _sage_update(p_ref, h_ref, wa_ref, ba_ref, hdim)
        oh_ref[...] = hnew
        om_ref[...] = jnp.maximum(
            jnp.dot(hnew, wl_ref[...],
                    preferred_element_type=jnp.float32) + bl_ref[...], 0.0)

    return pl.pallas_call(
        body,
        out_shape=(jax.ShapeDtypeStruct((n, hdim), jnp.float32),
                   jax.ShapeDtypeStruct((n, hdim), jnp.float32)),
    )(parts, h_prev, W_agg, b_agg.reshape(1, -1), W_lin, b_lin.reshape(1, -1))


def _tc_final_uv(parts, h_prev, W_agg, b_agg, mp_W1, mp_W2):
    n, hdim = h_prev.shape
    o = mp_W2.shape[1]

    def body(p_ref, h_ref, wa_ref, ba_ref, w1_ref, w2_ref, o_ref):
        h2 = _sage_update(p_ref, h_ref, wa_ref, ba_ref, hdim)
        w2 = w2_ref[...]
        wu = jnp.dot(w1_ref[0:hdim], w2, preferred_element_type=jnp.float32)
        wv = jnp.dot(w1_ref[hdim:2 * hdim], w2,
                     preferred_element_type=jnp.float32)
        wc = jnp.concatenate(
            [wu, jnp.zeros((hdim, 16 - o), jnp.float32),
             wv, jnp.zeros((hdim, 128 - 16 - o), jnp.float32)], axis=1)
        o_ref[...] = jnp.dot(h2, wc, preferred_element_type=jnp.float32)

    return pl.pallas_call(
        body, out_shape=jax.ShapeDtypeStruct((n, 128), jnp.float32),
    )(parts, h_prev, W_agg, b_agg.reshape(1, -1), mp_W1, mp_W2)


def _tc_logits(gs, gd, mp_b1, mp_W2, mp_b2):
    q = gs.shape[0]
    o = mp_W2.shape[1]

    def body(gs_ref, gd_ref, b1_ref, w2_ref, b2_ref, o_ref):
        c = jnp.dot(b1_ref[...], w2_ref[...],
                    preferred_element_type=jnp.float32) + b2_ref[...]
        z = gs_ref[:, 0:o] + gd_ref[:, 16:16 + o] + c
        mx = jnp.max(z, axis=1, keepdims=True)
        lse = mx + jnp.log(jnp.sum(jnp.exp(z - mx), axis=1, keepdims=True))
        o_ref[...] = z - lse

    return pl.pallas_call(
        body, out_shape=jax.ShapeDtypeStruct((q, o), jnp.float32),
    )(gs, gd, mp_b1.reshape(1, -1), mp_W2, mp_b2.reshape(1, -1))


def kernel(x, edge_index, batch, eval_edges,
           lin_W0, lin_b0, agg_W0, agg_b0,
           lin_W1, lin_b1, agg_W1, agg_b1,
           mp_W1, mp_b1, mp_W2, mp_b2):
    n = x.shape[0]
    e = edge_index.shape[1]
    src = edge_index[0].astype(jnp.int32)
    dst = edge_index[1].astype(jnp.int32)

    grp = _NW * _C
    e_pad = ((e + grp - 1) // grp) * grp
    pad = e_pad - e
    grp_n = _NS * 128
    n_pad = ((n + grp_n - 1) // grp_n) * grp_n
    if pad:
        src = jnp.concatenate([src, jnp.zeros((pad,), jnp.int32)])
        fill = n + jnp.arange(pad, dtype=jnp.int32) % (n_pad - n)
        dst = jnp.concatenate([dst, fill])
    src3 = src.reshape(_NW, -1, _C)
    dst3 = dst.reshape(_NW, -1, _C)

    m0 = _tc_node_linear(x, lin_W0, lin_b0)
    parts0 = _sc_segment_sum(m0, src3, dst3, n_pad)
    h1, m1 = _tc_update_and_msg(parts0, x, agg_W0, agg_b0, lin_W1, lin_b1)
    parts1 = _sc_segment_sum(m1, src3, dst3, n_pad)
    uv = _tc_final_uv(parts1, h1, agg_W1, agg_b1, mp_W1, mp_W2)
    es = eval_edges[0].astype(jnp.int32)
    ed = eval_edges[1].astype(jnp.int32)
    gs, gd = _sc_gather_pairs(uv, es, ed)
    return _tc_logits(gs, gd, mp_b1, mp_W2, mp_b2)
